# baseline (speedup 1.0000x reference)
import functools

import jax
import jax.numpy as jnp
from jax import lax
from jax.experimental import pallas as pl
from jax.experimental.pallas import tpu as pltpu
from jax.experimental.pallas import tpu_sc as plsc

N = 10000
NODE_F = 128
EDGE_F = 16
HID = 64
HEADS = 4
E = 320000

NBR = 1000
NB = N // NBR

TILES = 16
BLK = 128
EL = E + N
NBLK_E = 162
EPT = NBLK_E * BLK
EP1 = EPT * TILES

NW = 32
NBLK_E2 = 79
EPW = NBLK_E2 * BLK
EP2 = EPW * NW
EB = 2048

NHALF = N // 2
RPT = 312
_CHUNKS = ((0, 128), (128, 128), (256, 56))
_CHUNKS15 = ((0, 128), (128, 128), (256, 64))

_MESH = dict(core_axis_name="c", subcore_axis_name="s", num_cores=2,
             num_subcores=16)
_SC_PARAMS = pltpu.CompilerParams(needs_layout_passes=False,
                                  use_tc_tiling_on_sc=False)


def _ln(v, g, b):
    mu = v.mean(-1, keepdims=True)
    var = ((v - mu) ** 2).mean(-1, keepdims=True)
    return (v - mu) / jnp.sqrt(var + 1e-5) * g + b


def _node_embed_body(x_ref, w_ref, b_ref, g_ref, be_ref, o_ref):
    h = jnp.maximum(x_ref[...] @ w_ref[...] + b_ref[...], 0.0)
    o_ref[...] = _ln(h, g_ref[...], be_ref[...])


def _edge_embed_body(ea_ref, w_ref, b_ref, g_ref, be_ref, wc_ref, o_ref):
    f = jnp.maximum(ea_ref[...] @ w_ref[...] + b_ref[...], 0.0)
    o_ref[...] = _ln(f, g_ref[...], be_ref[...]) @ wc_ref[...]


def _pre_gat_tail(heads, f_out, xl, out_refs, c_ref, acc_ref, As, Ad):
    nq = heads
    fq = f_out // nq
    xl_refs = out_refs[:nq]
    a_refs = out_refs[nq:]
    i = pl.program_id(0)
    nb = pl.num_programs(0)
    for q in range(nq):
        xl_refs[q][...] = xl[:, q * fq:(q + 1) * fq]
    a_s = xl @ As
    a_d = xl @ Ad
    as_ref, ad_ref = a_refs
    as_ref[...] = a_s
    ad_ref[...] = a_d
    pad = jnp.full((1, 128 - heads), -1e30, jnp.float32)
    bs = jnp.concatenate([jnp.max(a_s, 0, keepdims=True), pad], 1)
    bd = jnp.concatenate([jnp.max(a_d, 0, keepdims=True), pad], 1)

    @pl.when(i == 0)
    def _():
        acc_ref[...] = jnp.full((8, 128), -1e30, jnp.float32)
        c_ref[...] = jnp.zeros((1, 128), jnp.float32)

    acc_ref[0:1, :] = jnp.maximum(acc_ref[0:1, :], bs)
    acc_ref[1:2, :] = jnp.maximum(acc_ref[1:2, :], bd)

    @pl.when(i == nb - 1)
    def _():
        t = acc_ref[0:1, :] + acc_ref[1:2, :]
        c_ref[...] = jnp.maximum(t, 0.2 * t)


def _make_pre_gat0_body(heads, f_out):
    def body(h_ref, w_ref, As_ref, Ad_ref, *refs):
        out_refs, c_ref, acc_ref = refs[:-2], refs[-2], refs[-1]
        xl = h_ref[...] @ w_ref[...]
        _pre_gat_tail(heads, f_out, xl, out_refs, c_ref, acc_ref,
                      As_ref[...], Ad_ref[...])
    return body


def _make_post_pre_body(heads, f_out):
    def body(*refs):
        n_refs = refs[:4]
        bb_ref, w_ref, As_ref, Ad_ref = refs[4:8]
        out_refs, c_ref, acc_ref = refs[8:-2], refs[-2], refs[-1]
        parts = []
        for q in range(4):
            nb = n_refs[q][...]
            parts.append(nb[:, :64] / (nb[:, 64 + q:65 + q] + 1e-16))
        h = jnp.concatenate(parts, 1) + bb_ref[...]
        h = jnp.where(h > 0, h, jnp.exp(h) - 1.0)
        xl = h @ w_ref[...]
        _pre_gat_tail(heads, f_out, xl, out_refs, c_ref, acc_ref,
                      As_ref[...], Ad_ref[...])
    return body


def _final_body(n0_ref, bb_ref, wn1_ref, bn1_ref, wn2_ref,
                bn2_ref, wa_ref, wb_ref,
                h_ref, np_ref, u1_ref, u2_ref, g_ref, acc_ref):
    i = pl.program_id(0)
    nb = pl.num_programs(0)
    n0 = n0_ref[...]
    h = n0[:, :64] / (n0[:, 64:65] + 1e-16) + bb_ref[...]
    h_ref[...] = h
    nh = jnp.maximum(h @ wn1_ref[...] + bn1_ref[...], 0.0)
    np_ref[...] = nh @ wn2_ref[...] + bn2_ref[...]
    u1_ref[...] = h @ wa_ref[...]
    u2_ref[...] = h @ wb_ref[...]
    zpad = jnp.zeros((1, 64), jnp.float32)
    mpad = jnp.full((1, 64), -1e30, jnp.float32)
    hs = jnp.concatenate([jnp.sum(h, 0, keepdims=True), zpad], 1)
    hm = jnp.concatenate([jnp.max(h, 0, keepdims=True), mpad], 1)

    @pl.when(i == 0)
    def _():
        acc_ref[0:1, :] = jnp.zeros((1, 128), jnp.float32)
        acc_ref[1:2, :] = jnp.full((1, 128), -1e30, jnp.float32)
        g_ref[...] = jnp.zeros((1, 128), jnp.float32)

    acc_ref[0:1, :] = acc_ref[0:1, :] + hs
    acc_ref[1:2, :] = jnp.maximum(acc_ref[1:2, :], hm)

    @pl.when(i == nb - 1)
    def _():
        g_ref[...] = jnp.concatenate(
            [acc_ref[0:1, :64] / float(N), acc_ref[1:2, :64]], 1)


def _edge_mlp_body(s12_ref, u3_ref, bc1_ref, wc2_ref, bc2_ref, wc3_ref,
                   bc3_ref, o_ref):
    z = jnp.maximum(s12_ref[...] + u3_ref[...] + bc1_ref[...], 0.0)
    e2 = jnp.maximum(z @ wc2_ref[...] + bc2_ref[...], 0.0)
    o_ref[...] = e2 @ wc3_ref[...] + bc3_ref[...]


def _make_gat_sc(heads):
    fq = 64
    fw = fq + 16
    cpq = fq // 16

    mesh = plsc.VectorSubcoreMesh(**_MESH)
    out_type = [jax.ShapeDtypeStruct((N, fw), jnp.float32)
                for _ in range(heads)]
    scratch = [
        pltpu.VMEM((N * heads,), jnp.float32),
        pltpu.VMEM((N * heads,), jnp.float32),
        pltpu.VMEM((16,), jnp.float32),
        pltpu.VMEM((BLK,), jnp.int32),
        pltpu.VMEM((BLK,), jnp.int32),
        pltpu.VMEM((BLK, fq), jnp.float32),
        pltpu.VMEM((BLK, fw), jnp.float32),
        pltpu.VMEM_SHARED((NHALF, fw), jnp.float32),
        pltpu.SemaphoreType.DMA,
    ]

    @functools.partial(pl.kernel, mesh=mesh, out_type=out_type,
                       scratch_types=scratch, compiler_params=_SC_PARAMS)
    def k(*refs):
        xls = refs[:heads]
        as_h, ad_h, c_hbm, s_hbm, d_hbm = refs[heads:heads + 5]
        num_os = refs[heads + 5:2 * heads + 5]
        (asv, adv, cv, sbuf, dlbuf, gbuf, rowbuf, acc,
         sem) = refs[2 * heads + 5:]

        cid = lax.axis_index("c")
        tid = lax.axis_index("s")
        iota = lax.iota(jnp.int32, 16)
        zf = jnp.zeros((16,), jnp.float32)
        rbase = cid * NHALF

        pltpu.sync_copy(as_h, asv)
        pltpu.sync_copy(ad_h, adv)
        pltpu.sync_copy(c_hbm, cv)

        def _zero_rowbuf():
            def _zb(e, carry):
                for kk in range(fw // 16):
                    rowbuf[e, pl.ds(kk * 16, 16)] = zf
                return carry
            lax.fori_loop(0, BLK, _zb, 0)

        _zero_rowbuf()

        def _acc_sweep(dst_sets):
            @pl.when(tid < 15)
            def _():
                for off, cs in _CHUNKS:
                    dst_sets(tid * RPT + off, cs)

            @pl.when(tid == 15)
            def _():
                for off, cs in _CHUNKS15:
                    dst_sets(15 * RPT + off, cs)

        _acc_sweep(lambda o, cs: pltpu.sync_copy(
            rowbuf.at[pl.ds(0, cs)], acc.at[pl.ds(o, cs)]))
        plsc.subcore_barrier()

        for p in range(heads):
            hv = jnp.full((16,), p, jnp.int32)
            exlane = jnp.full((16,), fq + p, jnp.int32)
            ct = plsc.load_gather(cv, [hv])

            def block_body(b, carry, p=p, hv=hv, exlane=exlane, ct=ct):
                ebase = tid * EPT + b * BLK
                pltpu.sync_copy(s_hbm.at[pl.ds(ebase, BLK)], sbuf)
                pltpu.async_copy(xls[p].at[sbuf], gbuf, sem).wait()
                pltpu.sync_copy(d_hbm.at[pl.ds(ebase, BLK)], dlbuf)

                for j in range(BLK // 16):
                    ev = j * 16 + iota
                    srcv = sbuf[pl.ds(j * 16, 16)]
                    dstv = dlbuf[pl.ds(j * 16, 16)]
                    if heads == 4:
                        av = plsc.load_gather(asv, [srcv * 4 + hv])
                        dv = plsc.load_gather(adv, [dstv * 4 + hv])
                    else:
                        av = plsc.load_gather(asv, [srcv])
                        dv = plsc.load_gather(adv, [dstv])
                    t = av + dv
                    alpha = jnp.maximum(t, 0.2 * t)
                    ex = jnp.exp(alpha - ct)
                    dloc = dstv - cid * NHALF
                    ok = ((dloc >= 0) & (dloc < NHALF)
                          & (ebase + ev < EL))
                    ex = jnp.where(ok, ex, 0.0)
                    dlbuf[pl.ds(j * 16, 16)] = jnp.where(ok, dloc, 0)
                    plsc.store_scatter(rowbuf, [ev, exlane], ex)

                def scale_body(e, c2):
                    e_v = jnp.zeros((16,), jnp.int32) + e
                    sp = plsc.load_gather(rowbuf, [e_v, exlane])
                    for kk in range(cpq):
                        rowbuf[e, pl.ds(kk * 16, 16)] = (
                            gbuf[e, pl.ds(kk * 16, 16)] * sp)
                    return c2

                lax.fori_loop(0, BLK, scale_body, 0)
                pltpu.sync_copy(rowbuf, acc.at[dlbuf], add=True)
                return carry

            lax.fori_loop(0, NBLK_E, block_body, 0)
            plsc.subcore_barrier()

            _acc_sweep(lambda o, cs, p=p: pltpu.sync_copy(
                acc.at[pl.ds(o, cs)], num_os[p].at[pl.ds(rbase + o, cs)]))
            if p < heads - 1:
                plsc.subcore_barrier()
                _zero_rowbuf()
                _acc_sweep(lambda o, cs: pltpu.sync_copy(
                    rowbuf.at[pl.ds(0, cs)], acc.at[pl.ds(o, cs)]))
                plsc.subcore_barrier()

    return k


_gat_sc_4 = _make_gat_sc(4)
_gat_sc_1 = _make_gat_sc(1)


def _make_edge_gather():
    mesh = plsc.VectorSubcoreMesh(**_MESH)
    out_type = jax.ShapeDtypeStruct((EP2, HID), jnp.float32)
    scratch = [
        pltpu.VMEM((BLK,), jnp.int32),
        pltpu.VMEM((BLK,), jnp.int32),
        pltpu.VMEM((BLK, HID), jnp.float32),
        pltpu.VMEM((BLK, HID), jnp.float32),
        pltpu.SemaphoreType.DMA,
    ]

    @functools.partial(pl.kernel, mesh=mesh, out_type=out_type,
                       scratch_types=scratch, compiler_params=_SC_PARAMS)
    def k(u1, u2, s_hbm, d_hbm, s12_o, sbuf, dbuf, r1, r2, sem):
        cid = lax.axis_index("c")
        tid = lax.axis_index("s")
        w = tid * 2 + cid

        def body(b, carry):
            off = w * EPW + b * BLK
            pltpu.sync_copy(s_hbm.at[pl.ds(off, BLK)], sbuf)
            pltpu.sync_copy(d_hbm.at[pl.ds(off, BLK)], dbuf)
            pltpu.async_copy(u1.at[sbuf], r1, sem).wait()
            pltpu.async_copy(u2.at[dbuf], r2, sem).wait()

            def add_body(e, c2):
                for kk in range(HID // 16):
                    r1[e, pl.ds(kk * 16, 16)] = (
                        r1[e, pl.ds(kk * 16, 16)] + r2[e, pl.ds(kk * 16, 16)])
                return c2

            lax.fori_loop(0, BLK, add_body, 0)
            pltpu.sync_copy(r1, s12_o.at[pl.ds(off, BLK)])
            return carry

        lax.fori_loop(0, NBLK_E2, body, 0)

    return k


_edge_gather_sc = _make_edge_gather()


def _full(shape):
    nd = len(shape)
    return pl.BlockSpec(shape, lambda i, _n=nd: (0,) * _n)


def _node_embed(x, w, b, g, be):
    return pl.pallas_call(
        _node_embed_body,
        grid=(NB,),
        in_specs=[pl.BlockSpec((NBR, NODE_F), lambda i: (i, 0)),
                  _full((NODE_F, HID)), _full((1, HID)), _full((1, HID)),
                  _full((1, HID))],
        out_specs=pl.BlockSpec((NBR, HID), lambda i: (i, 0)),
        out_shape=jax.ShapeDtypeStruct((N, HID), jnp.float32),
    )(x, w, b, g, be)


def _edge_embed(ea, w, b, g, be, wc):
    return pl.pallas_call(
        _edge_embed_body,
        grid=(EP2 // EB,),
        in_specs=[pl.BlockSpec((EB, EDGE_F), lambda i: (i, 0)),
                  _full((EDGE_F, HID // 2)), _full((1, HID // 2)),
                  _full((1, HID // 2)), _full((1, HID // 2)),
                  _full((HID // 2, HID))],
        out_specs=pl.BlockSpec((EB, HID), lambda i: (i, 0)),
        out_shape=jax.ShapeDtypeStruct((EP2, HID), jnp.float32),
    )(ea, w, b, g, be, wc)


def _pre_outs(heads, f_out):
    nq = heads
    fq = f_out // nq
    shapes, specs = [], []
    for _ in range(nq):
        shapes.append(jax.ShapeDtypeStruct((N, fq), jnp.float32))
        specs.append(pl.BlockSpec((NBR, fq), lambda i, _f=fq: (i, 0)))
    for _ in range(2):
        shapes.append(jax.ShapeDtypeStruct((N, heads), jnp.float32))
        specs.append(pl.BlockSpec((NBR, heads), lambda i: (i, 0)))
    shapes.append(jax.ShapeDtypeStruct((1, 128), jnp.float32))
    specs.append(pl.BlockSpec((1, 128), lambda i: (0, 0)))
    return shapes, specs


def _pre_gat0(h0, w, As, Ad, heads, f_out):
    shapes, specs = _pre_outs(heads, f_out)
    return pl.pallas_call(
        _make_pre_gat0_body(heads, f_out),
        grid=(NB,),
        in_specs=[pl.BlockSpec((NBR, HID), lambda i: (i, 0)),
                  _full((HID, f_out)), _full((f_out, heads)),
                  _full((f_out, heads))],
        out_specs=specs,
        out_shape=shapes,
        scratch_shapes=[pltpu.VMEM((8, 128), jnp.float32)],
    )(h0, w, As, Ad)


def _post_pre(nums, bb, w, As, Ad, heads, f_out):
    shapes, specs = _pre_outs(heads, f_out)
    return pl.pallas_call(
        _make_post_pre_body(heads, f_out),
        grid=(NB,),
        in_specs=[pl.BlockSpec((NBR, 80), lambda i: (i, 0))
                  for _ in range(4)] +
                 [_full((1, 256)), _full((256, f_out)),
                  _full((f_out, heads)), _full((f_out, heads))],
        out_specs=specs,
        out_shape=shapes,
        scratch_shapes=[pltpu.VMEM((8, 128), jnp.float32)],
    )(*nums, bb, w, As, Ad)


def _final(num0, bb2, wn1, bn1, wn2, bn2, wa, wb):
    return pl.pallas_call(
        _final_body,
        grid=(NB,),
        in_specs=[pl.BlockSpec((NBR, 80), lambda i: (i, 0)),
                  _full((1, HID)), _full((HID, HID)), _full((1, HID)),
                  _full((HID, 1)), _full((1, 1)),
                  _full((HID, HID)), _full((HID, HID))],
        out_specs=[pl.BlockSpec((NBR, HID), lambda i: (i, 0)),
                   pl.BlockSpec((NBR, 1), lambda i: (i, 0)),
                   pl.BlockSpec((NBR, HID), lambda i: (i, 0)),
                   pl.BlockSpec((NBR, HID), lambda i: (i, 0)),
                   pl.BlockSpec((1, 128), lambda i: (0, 0))],
        out_shape=[jax.ShapeDtypeStruct((N, HID), jnp.float32),
                   jax.ShapeDtypeStruct((N, 1), jnp.float32),
                   jax.ShapeDtypeStruct((N, HID), jnp.float32),
                   jax.ShapeDtypeStruct((N, HID), jnp.float32),
                   jax.ShapeDtypeStruct((1, 128), jnp.float32)],
        scratch_shapes=[pltpu.VMEM((8, 128), jnp.float32)],
    )(num0, bb2, wn1, bn1, wn2, bn2, wa, wb)


def _edge_mlp(s12, u3, bc1, wc2, bc2, wc3, bc3):
    return pl.pallas_call(
        _edge_mlp_body,
        grid=(EP2 // EB,),
        in_specs=[pl.BlockSpec((EB, HID), lambda i: (i, 0)),
                  pl.BlockSpec((EB, HID), lambda i: (i, 0)),
                  _full((1, HID)), _full((HID, HID)), _full((1, HID)),
                  _full((HID, 1)), _full((1, 1))],
        out_specs=pl.BlockSpec((EB, 1), lambda i: (i, 0)),
        out_shape=jax.ShapeDtypeStruct((EP2, 1), jnp.float32),
    )(s12, u3, bc1, wc2, bc2, wc3, bc3)


def _att_mat(att):
    h = att.shape[0]
    eye = jnp.eye(h, dtype=jnp.float32)
    return (eye[:, None, :] * att[:, :, None]).reshape(h * att.shape[1], h)


def _row(v):
    return v.reshape(1, -1)


def kernel(x, edge_attr, edge_index, params):
    p = params
    src = edge_index[0].astype(jnp.int32)
    dst = edge_index[1].astype(jnp.int32)
    loop = jnp.arange(N, dtype=jnp.int32)
    s_pad = jnp.concatenate([src, loop, jnp.zeros((EP1 - EL,), jnp.int32)])
    d_pad = jnp.concatenate([dst, loop, jnp.zeros((EP1 - EL,), jnp.int32)])
    s2 = jnp.concatenate([src, jnp.zeros((EP2 - E,), jnp.int32)])
    d2 = jnp.concatenate([dst, jnp.zeros((EP2 - E,), jnp.int32)])
    ea_pad = jnp.pad(edge_attr, ((0, EP2 - E), (0, 0)))

    h0 = _node_embed(x, p['W_ne'], _row(p['b_ne']), _row(p['g_ne']),
                     _row(p['be_ne']))
    u3 = _edge_embed(ea_pad, p['W_ee'], _row(p['b_ee']), _row(p['g_ee']),
                     _row(p['be_ee']), p['Wc1'][2 * HID:])

    out0 = _pre_gat0(h0, p['W0'], _att_mat(p['as0']), _att_mat(p['ad0']),
                     4, 256)
    qs, (as0t, ad0t, c0) = out0[:4], out0[4:]
    nums0 = _gat_sc_4(*qs, as0t.reshape(-1), ad0t.reshape(-1),
                      c0[0, :16], s_pad, d_pad)

    out1 = _post_pre(nums0, _row(p['bb0']), p['W1'],
                     _att_mat(p['as1']), _att_mat(p['ad1']), 4, 256)
    qs, (as1t, ad1t, c1) = out1[:4], out1[4:]
    nums1 = _gat_sc_4(*qs, as1t.reshape(-1), ad1t.reshape(-1),
                      c1[0, :16], s_pad, d_pad)

    xt2, as2t, ad2t, c2 = _post_pre(
        nums1, _row(p['bb1']), p['W2'],
        _att_mat(p['as2']), _att_mat(p['ad2']), 1, 64)
    (m2,) = _gat_sc_1(xt2, as2t.reshape(-1), ad2t.reshape(-1),
                      c2[0, :16], s_pad, d_pad)

    h_out, np_out, u1, u2, graph_feat = _final(
        m2, _row(p['bb2']), p['Wn1'], _row(p['bn1']),
        p['Wn2'], _row(p['bn2']), p['Wc1'][:HID], p['Wc1'][HID:2 * HID])

    s12 = _edge_gather_sc(u1, u2, s2, d2)
    ep = _edge_mlp(s12, u3, _row(p['bc1']), p['Wc2'], _row(p['bc2']),
                   p['Wc3'], _row(p['bc3']))

    edge_pred = ep[:E, 0]
    node_pred = np_out[:, 0]
    return (edge_pred, node_pred, graph_feat, h_out)
